# baseline (speedup 1.0000x reference)
import functools

import jax
import jax.numpy as jnp
from jax import lax
from jax.experimental import pallas as pl
from jax.experimental.pallas import tpu as pltpu
from jax.experimental.pallas import tpu_sc as plsc

NC, NS, L = 2, 16, 16
NW = NC * NS
B, C, T = 1024, 1000, 10
ROWS_PER_W = B // NW
NG = 4
TC_FULL = C // 128
TAIL = C - TC_FULL * 128
TAIL_CHUNKS = (TAIL + L - 1) // L
CP = TC_FULL * 128 + TAIL_CHUNKS * L
NEG = -1e30

_mesh = plsc.VectorSubcoreMesh(
    core_axis_name="c", subcore_axis_name="s", num_cores=NC, num_subcores=NS
)


@functools.partial(
    pl.kernel,
    out_type=(
        jax.ShapeDtypeStruct((NW, L), jnp.float32),
        jax.ShapeDtypeStruct((NW, L), jnp.float32),
    ),
    mesh=_mesh,
    compiler_params=pltpu.CompilerParams(
        needs_layout_passes=False, use_tc_tiling_on_sc=True
    ),
    scratch_types=[
        pltpu.VMEM((NG * 8, 8, 128), jnp.float32),
        pltpu.VMEM((NG * 8, 8, 128), jnp.float32),
        pltpu.VMEM((ROWS_PER_W, T), jnp.int32),
        pltpu.VMEM((L,), jnp.float32),
        pltpu.VMEM((L,), jnp.float32),
        pltpu.SemaphoreType.DMA((NG,)),
    ],
)
def _loss_kernel(x_hbm, m_hbm, xt_hbm, mt_hbm, tgt_hbm, out_a, out_t,
                 xb, mb, tbuf, avec, tvec, sems):
    wid = lax.axis_index("s") * NC + lax.axis_index("c")
    base_row = wid * ROWS_PER_W

    for g in range(NG):
        pltpu.sync_copy(
            tgt_hbm.at[pl.ds(base_row + 8 * g, 8), pl.ds(0, T)],
            tbuf.at[pl.ds(8 * g, 8)],
        )

    for g in range(NG):
        r0 = base_row + 8 * g
        for c in range(TC_FULL):
            pltpu.async_copy(
                x_hbm.at[pl.ds(r0, 8), pl.ds(128 * c, 128)],
                xb.at[8 * g + c], sems.at[g])
            pltpu.async_copy(
                m_hbm.at[pl.ds(r0, 8), pl.ds(128 * c, 128)],
                mb.at[8 * g + c], sems.at[g])
        pltpu.async_copy(
            xt_hbm.at[pl.ds(r0, 8), pl.ds(0, 128)],
            xb.at[8 * g + TC_FULL], sems.at[g])
        pltpu.async_copy(
            mt_hbm.at[pl.ds(r0, 8), pl.ds(0, 128)],
            mb.at[8 * g + TC_FULL], sems.at[g])

    lane = lax.iota(jnp.int32, L)
    tail_keep = lane < (TAIL - (TAIL_CHUNKS - 1) * L)

    def group_body(g, carry):
        accs, thsum = carry
        accs = list(accs)

        for c in range(TC_FULL):
            pltpu.make_async_copy(
                x_hbm.at[pl.ds(0, 8), pl.ds(0, 128)], xb.at[0], sems.at[g]).wait()
            pltpu.make_async_copy(
                m_hbm.at[pl.ds(0, 8), pl.ds(0, 128)], mb.at[0], sems.at[g]).wait()
        pltpu.make_async_copy(
            xt_hbm.at[pl.ds(0, 8), pl.ds(0, 128)], xb.at[0], sems.at[g]).wait()
        pltpu.make_async_copy(
            mt_hbm.at[pl.ds(0, 8), pl.ds(0, 128)], mb.at[0], sems.at[g]).wait()

        for r in range(8):
            thetas = []
            for t in range(T):
                tg = plsc.load_gather(
                    tbuf,
                    [jnp.full((L,), 8 * g + r, jnp.int32),
                     jnp.full((L,), t, jnp.int32)],
                )
                tile = 8 * g + jnp.right_shift(tg, 7)
                col = jnp.bitwise_and(tg, 127)
                rvec = jnp.full((L,), r, jnp.int32)
                xt = plsc.load_gather(xb, [tile, rvec, col])
                mt = plsc.load_gather(mb, [tile, rvec, col])
                th = xt - mt - 1.0
                thetas.append(th)
                thsum = thsum + th

            for c in range(TC_FULL + 1):
                nk = 8 if c < TC_FULL else TAIL_CHUNKS
                for k in range(nk):
                    s = (xb[8 * g + c, r, pl.ds(k * L, L)]
                         - mb[8 * g + c, r, pl.ds(k * L, L)])
                    if c == TC_FULL and k == nk - 1:
                        s = jnp.where(tail_keep, s, NEG)
                    for t in range(T):
                        accs[t] = accs[t] + jnp.maximum(s, thetas[t])
        return tuple(accs), thsum

    zero = jnp.zeros((L,), jnp.float32)
    accs, thsum = lax.fori_loop(
        0, NG, group_body, (tuple(zero for _ in range(T)), zero)
    )

    acc = accs[0]
    for t in range(1, T):
        acc = acc + accs[t]

    avec[...] = acc
    tvec[...] = thsum
    pltpu.sync_copy(avec, out_a.at[wid])
    pltpu.sync_copy(tvec, out_t.at[wid])


def kernel(input_data, target, adaptive_margin):
    pad = ((0, 0), (0, 128 - TAIL))
    x_tail = jnp.pad(input_data[:, 128 * TC_FULL:], pad)
    m_tail = jnp.pad(adaptive_margin[:, 128 * TC_FULL:], pad)
    out_a, out_t = _loss_kernel(
        input_data, adaptive_margin, x_tail, m_tail, target.astype(jnp.int32)
    )
    total = jnp.sum(out_a) - CP * jnp.sum(out_t[:, 0]) - jnp.float32(B * T)
    return total / jnp.float32(C)


# speedup vs baseline: 1.3485x; 1.3485x over previous
import functools

import jax
import jax.numpy as jnp
from jax import lax
from jax.experimental import pallas as pl
from jax.experimental.pallas import tpu as pltpu
from jax.experimental.pallas import tpu_sc as plsc

NC, NS, L = 2, 16, 16
NW = NC * NS
B, C, T = 1024, 1000, 10
NTILES = C // 8
TPW = 31
NSB = 128 // L

_mesh = plsc.VectorSubcoreMesh(
    core_axis_name="c", subcore_axis_name="s", num_cores=NC, num_subcores=NS
)


@functools.partial(
    pl.kernel,
    out_type=(
        jax.ShapeDtypeStruct((NW, L), jnp.float32),
        jax.ShapeDtypeStruct((NW, L), jnp.float32),
    ),
    mesh=_mesh,
    compiler_params=pltpu.CompilerParams(
        needs_layout_passes=False, use_tc_tiling_on_sc=True
    ),
    scratch_types=[
        pltpu.VMEM((32, 8, 128), jnp.float32),
        pltpu.VMEM((32, 8, 128), jnp.float32),
        pltpu.VMEM((2, 8, 128), jnp.float32),
        pltpu.VMEM((2, 8, 128), jnp.float32),
        pltpu.VMEM((T, 128), jnp.float32),
        pltpu.VMEM((T, 128), jnp.float32),
        pltpu.VMEM((T, 128), jnp.int32),
        pltpu.VMEM((L,), jnp.int32),
        pltpu.VMEM((T, L), jnp.float32),
        pltpu.VMEM((L,), jnp.float32),
        pltpu.VMEM((L,), jnp.float32),
        pltpu.VMEM_SHARED((4 * T, 128), jnp.float32),
        pltpu.SemaphoreType.DMA,
        pltpu.SemaphoreType.DMA,
        pltpu.SemaphoreType.DMA,
    ],
)
def _loss_kernel(xt_hbm, mt_hbm, tgt_hbm, out_a, out_t,
                 xa, ma, xb, mb, thb, pth, tbuf, idxb, accv, avec, tvec,
                 shared, semA, sem0, sem1):
    c = lax.axis_index("c")
    s = lax.axis_index("s")
    wid = s * NC + c
    tc = c * 4 + jnp.bitwise_and(s, 3)
    tcl = jnp.bitwise_and(s, 3)
    q = jnp.right_shift(s, 2)
    tbase = TPW * q
    tlast = tbase + TPW - 1
    col0 = 128 * tc

    for i in range(TPW):
        tr = tbase + i
        pltpu.async_copy(
            xt_hbm.at[pl.ds(8 * tr, 8), pl.ds(col0, 128)], xa.at[i], semA)
        pltpu.async_copy(
            mt_hbm.at[pl.ds(8 * tr, 8), pl.ds(col0, 128)], ma.at[i], semA)
    t32 = jnp.where(q == 0, NTILES - 1, tlast)
    pltpu.async_copy(
        xt_hbm.at[pl.ds(8 * t32, 8), pl.ds(col0, 128)], xa.at[TPW], semA)
    pltpu.async_copy(
        mt_hbm.at[pl.ds(8 * t32, 8), pl.ds(col0, 128)], ma.at[TPW], semA)

    pltpu.sync_copy(tgt_hbm.at[pl.ds(0, 8), pl.ds(col0, 128)], tbuf.at[pl.ds(0, 8)])
    pltpu.sync_copy(tgt_hbm.at[pl.ds(8, 2), pl.ds(col0, 128)], tbuf.at[pl.ds(8, 2)])

    zero = jnp.zeros((L,), jnp.float32)
    for t in range(T):
        thb[t, pl.ds(0, L)] = zero
        for sb in range(1, NSB):
            thb[t, pl.ds(L * sb, L)] = zero

    for i in range(TPW + 1):
        pltpu.make_async_copy(
            xt_hbm.at[pl.ds(0, 8), pl.ds(0, 128)], xa.at[i], semA).wait()
        pltpu.make_async_copy(
            mt_hbm.at[pl.ds(0, 8), pl.ds(0, 128)], ma.at[i], semA).wait()

    lanes = lax.iota(jnp.int32, L)
    psum = zero
    for t in range(T):
        for sb in range(NSB):
            tg = tbuf[t, pl.ds(L * sb, L)]
            tr = jnp.right_shift(tg, 3)
            tl = tr - tbase
            extra = jnp.logical_and(tr == NTILES - 1, q == 0)
            valid = jnp.logical_or(
                jnp.logical_and(tl >= 0, tl < TPW), extra)
            idx = jnp.clip(jnp.where(extra, TPW, tl), 0, TPW)
            row = jnp.bitwise_and(tg, 7)
            col = lanes + (L * sb)
            xg = plsc.load_gather(xa, [idx, row, col])
            mg = plsc.load_gather(ma, [idx, row, col])
            g = xg - mg
            gval = jnp.where(valid, g, 0.0)
            psum = psum + gval
            pth[t, pl.ds(L * sb, L)] = jnp.where(valid, g - 1.0, 0.0)

    idxb[...] = lanes + T * tcl

    @pl.when(q == 0)
    def _():
        pltpu.sync_copy(thb, shared.at[pl.ds(T * tcl, T)])
    plsc.subcore_barrier()
    pltpu.sync_copy(pth, shared.at[idxb.at[pl.ds(0, T)]], add=True)
    plsc.subcore_barrier()
    pltpu.sync_copy(shared.at[pl.ds(T * tcl, T)], thb)

    for t in range(T):
        accv[t] = zero

    def fire(i, slot, sem):
        tr = tbase + i
        pltpu.async_copy(
            xt_hbm.at[pl.ds(8 * tr, 8), pl.ds(col0, 128)], xb.at[slot], sem)
        pltpu.async_copy(
            mt_hbm.at[pl.ds(8 * tr, 8), pl.ds(col0, 128)], mb.at[slot], sem)

    def wait(slot, sem):
        pltpu.make_async_copy(
            xt_hbm.at[pl.ds(0, 8), pl.ds(0, 128)], xb.at[slot], sem).wait()
        pltpu.make_async_copy(
            mt_hbm.at[pl.ds(0, 8), pl.ds(0, 128)], mb.at[slot], sem).wait()

    def tile_compute(slot, xr, mr):
        accs = [accv[t] for t in range(T)]
        for sb in range(NSB):
            ths = [thb[t, pl.ds(L * sb, L)] for t in range(T)]
            for r in range(8):
                s_ = xr[slot, r, pl.ds(L * sb, L)] - mr[slot, r, pl.ds(L * sb, L)]
                for t in range(T):
                    accs[t] = accs[t] + jnp.maximum(s_, ths[t])
        for t in range(T):
            accv[t] = accs[t]

    fire(0, 0, sem0)
    fire(1, 1, sem1)

    def pair_body(i, carry):
        i0 = 2 * i
        wait(0, sem0)
        tile_compute(0, xb, mb)
        fire(jnp.minimum(i0 + 2, TPW - 1), 0, sem0)
        wait(1, sem1)
        tile_compute(1, xb, mb)
        fire(jnp.minimum(i0 + 3, TPW - 1), 1, sem1)
        return carry

    lax.fori_loop(0, TPW // 2, pair_body, jnp.int32(0))

    wait(0, sem0)
    tile_compute(0, xb, mb)
    wait(1, sem1)

    @pl.when(q == 0)
    def _():
        tile_compute(TPW, xa, ma)

    acc = accv[0]
    for t in range(1, T):
        acc = acc + accv[t]
    avec[...] = acc
    tvec[...] = psum
    pltpu.sync_copy(avec, out_a.at[wid])
    pltpu.sync_copy(tvec, out_t.at[wid])


def kernel(input_data, target, adaptive_margin):
    out_a, out_t = _loss_kernel(
        input_data.T, adaptive_margin.T, target.astype(jnp.int32).T
    )
    bt = jnp.float32(B * T)
    total = jnp.sum(out_a) - jnp.float32(C) * (jnp.sum(out_t) - bt) - bt
    return total / jnp.float32(C)


# speedup vs baseline: 1.5203x; 1.1274x over previous
import functools

import jax
import jax.numpy as jnp
from jax import lax
from jax.experimental import pallas as pl
from jax.experimental.pallas import tpu as pltpu
from jax.experimental.pallas import tpu_sc as plsc

NC, NS, L = 2, 16, 16
NW = NC * NS
B, C, T = 1024, 1000, 10
NTILES = C // 8
TPW = 31
NSB = 128 // L

_mesh = plsc.VectorSubcoreMesh(
    core_axis_name="c", subcore_axis_name="s", num_cores=NC, num_subcores=NS
)


@functools.partial(
    pl.kernel,
    out_type=jax.ShapeDtypeStruct((2 * NW, L), jnp.float32),
    mesh=_mesh,
    compiler_params=pltpu.CompilerParams(
        needs_layout_passes=False, use_tc_tiling_on_sc=True
    ),
    scratch_types=[
        pltpu.VMEM((32, 8, 128), jnp.float32),
        pltpu.VMEM((32, 8, 128), jnp.float32),
        pltpu.VMEM((2, 8, 128), jnp.float32),
        pltpu.VMEM((2, 8, 128), jnp.float32),
        pltpu.VMEM((T, 128), jnp.float32),
        pltpu.VMEM((T, 128), jnp.float32),
        pltpu.VMEM((T, 128), jnp.int32),
        pltpu.VMEM((L,), jnp.int32),
        pltpu.VMEM((T, L), jnp.float32),
        pltpu.VMEM((L,), jnp.float32),
        pltpu.VMEM((L,), jnp.float32),
        pltpu.VMEM_SHARED((4 * T, 128), jnp.float32),
        pltpu.SemaphoreType.DMA,
        pltpu.SemaphoreType.DMA,
    ],
)
def _loss_kernel(xt_hbm, mt_hbm, tgt_hbm, out,
                 xa, ma, xb, mb, thb, pth, tbuf, idxb, accv, avec, tvec,
                 shared, sem0, sem1):
    c = lax.axis_index("c")
    s = lax.axis_index("s")
    wid = s * NC + c
    tc = c * 4 + jnp.bitwise_and(s, 3)
    tcl = jnp.bitwise_and(s, 3)
    q = jnp.right_shift(s, 2)
    tbase = TPW * q
    col0 = 128 * tc

    for i in range(TPW):
        tr = tbase + i
        pltpu.async_copy(
            xt_hbm.at[pl.ds(8 * tr, 8), pl.ds(col0, 128)], xa.at[i], sem0)
        pltpu.async_copy(
            mt_hbm.at[pl.ds(8 * tr, 8), pl.ds(col0, 128)], ma.at[i], sem1)
    t32 = jnp.where(q == 0, NTILES - 1, tbase + TPW - 1)
    pltpu.async_copy(
        xt_hbm.at[pl.ds(8 * t32, 8), pl.ds(col0, 128)], xa.at[TPW], sem0)
    pltpu.async_copy(
        mt_hbm.at[pl.ds(8 * t32, 8), pl.ds(col0, 128)], ma.at[TPW], sem1)

    pltpu.sync_copy(tgt_hbm.at[pl.ds(0, 8), pl.ds(col0, 128)], tbuf.at[pl.ds(0, 8)])
    pltpu.sync_copy(tgt_hbm.at[pl.ds(8, 2), pl.ds(col0, 128)], tbuf.at[pl.ds(8, 2)])

    zero = jnp.zeros((L,), jnp.float32)
    for t in range(T):
        for sb in range(NSB):
            thb[t, pl.ds(L * sb, L)] = zero

    lanes = lax.iota(jnp.int32, L)
    idxb[...] = lanes + T * tcl

    def tgt_addr(t, sb):
        tg = tbuf[t, pl.ds(L * sb, L)]
        tl = jnp.right_shift(tg, 3) - tbase
        extra = jnp.logical_and(jnp.right_shift(tg, 3) == NTILES - 1, q == 0)
        valid = jnp.logical_or(jnp.logical_and(tl >= 0, tl < TPW), extra)
        idx = jnp.clip(jnp.where(extra, TPW, tl), 0, TPW)
        return idx, jnp.bitwise_and(tg, 7), lanes + L * sb, valid

    for i in range(TPW + 1):
        pltpu.make_async_copy(
            xt_hbm.at[pl.ds(0, 8), pl.ds(0, 128)], xa.at[i], sem0).wait()

    def xg_body(t, carry):
        for sb in range(NSB):
            idx, row, col, _ = tgt_addr(t, sb)
            pth[t, pl.ds(L * sb, L)] = plsc.load_gather(xa, [idx, row, col])
        return carry

    lax.fori_loop(0, T, xg_body, jnp.int32(0))

    for i in range(TPW + 1):
        pltpu.make_async_copy(
            mt_hbm.at[pl.ds(0, 8), pl.ds(0, 128)], ma.at[i], sem1).wait()

    def mg_body(t, psum):
        for sb in range(NSB):
            idx, row, col, valid = tgt_addr(t, sb)
            g = pth[t, pl.ds(L * sb, L)] - plsc.load_gather(ma, [idx, row, col])
            psum = psum + jnp.where(valid, g, 0.0)
            pth[t, pl.ds(L * sb, L)] = jnp.where(valid, g - 1.0, 0.0)
        return psum

    psum = lax.fori_loop(0, T, mg_body, zero)

    @pl.when(q == 0)
    def _():
        pltpu.sync_copy(thb, shared.at[pl.ds(T * tcl, T)])
    plsc.subcore_barrier()
    pltpu.sync_copy(pth, shared.at[idxb.at[pl.ds(0, T)]], add=True)
    plsc.subcore_barrier()
    pltpu.sync_copy(shared.at[pl.ds(T * tcl, T)], thb)

    for t in range(T):
        accv[t] = zero

    def fire(i, slot, sem):
        tr = tbase + i
        pltpu.async_copy(
            xt_hbm.at[pl.ds(8 * tr, 8), pl.ds(col0, 128)], xb.at[slot], sem)
        pltpu.async_copy(
            mt_hbm.at[pl.ds(8 * tr, 8), pl.ds(col0, 128)], mb.at[slot], sem)

    def wait(slot, sem):
        pltpu.make_async_copy(
            xt_hbm.at[pl.ds(0, 8), pl.ds(0, 128)], xb.at[slot], sem).wait()
        pltpu.make_async_copy(
            mt_hbm.at[pl.ds(0, 8), pl.ds(0, 128)], mb.at[slot], sem).wait()

    def tile_compute(slot, xr, mr):
        def sb_body(sb, accs):
            accs = list(accs)
            ths = [thb[t, pl.ds(L * sb, L)] for t in range(T)]
            for r in range(8):
                s_ = xr[slot, r, pl.ds(L * sb, L)] - mr[slot, r, pl.ds(L * sb, L)]
                for t in range(T):
                    accs[t] = accs[t] + jnp.maximum(s_, ths[t])
            return tuple(accs)

        accs = lax.fori_loop(
            0, NSB, sb_body, tuple(accv[t] for t in range(T)))
        for t in range(T):
            accv[t] = accs[t]

    fire(0, 0, sem0)
    fire(1, 1, sem1)

    def pair_body(i, carry):
        i0 = 2 * i
        wait(0, sem0)
        tile_compute(0, xb, mb)
        fire(jnp.minimum(i0 + 2, TPW - 1), 0, sem0)
        wait(1, sem1)
        tile_compute(1, xb, mb)
        fire(jnp.minimum(i0 + 3, TPW - 1), 1, sem1)
        return carry

    lax.fori_loop(0, TPW // 2, pair_body, jnp.int32(0))

    wait(0, sem0)
    tile_compute(0, xb, mb)
    wait(1, sem1)

    @pl.when(q == 0)
    def _():
        tile_compute(TPW, xa, ma)

    acc = accv[0]
    for t in range(1, T):
        acc = acc + accv[t]
    avec[...] = acc
    tvec[...] = psum
    pltpu.sync_copy(avec, out.at[wid])
    pltpu.sync_copy(tvec, out.at[NW + wid])


def kernel(input_data, target, adaptive_margin):
    out = _loss_kernel(
        input_data.T, adaptive_margin.T, target.astype(jnp.int32).T
    )
    bt = jnp.float32(B * T)
    total = (jnp.sum(out[:NW])
             - jnp.float32(C) * (jnp.sum(out[NW:]) - bt) - bt)
    return total / jnp.float32(C)


# speedup vs baseline: 1.8724x; 1.2316x over previous
import functools

import jax
import jax.numpy as jnp
from jax import lax
from jax.experimental import pallas as pl
from jax.experimental.pallas import tpu as pltpu
from jax.experimental.pallas import tpu_sc as plsc

NC, NS, L = 2, 16, 16
NW = NC * NS
B, C, T = 1024, 1000, 10
NTILES = C // 8
TPW = 31
NSB = 128 // L

_mesh = plsc.VectorSubcoreMesh(
    core_axis_name="c", subcore_axis_name="s", num_cores=NC, num_subcores=NS
)


@functools.partial(
    pl.kernel,
    out_type=jax.ShapeDtypeStruct((2 * NW, L), jnp.float32),
    mesh=_mesh,
    compiler_params=pltpu.CompilerParams(
        needs_layout_passes=False, use_tc_tiling_on_sc=True
    ),
    scratch_types=[
        pltpu.VMEM((256, 128), jnp.float32),
        pltpu.VMEM((256, 128), jnp.float32),
        pltpu.VMEM((T, 128), jnp.float32),
        pltpu.VMEM((T, 128), jnp.float32),
        pltpu.VMEM((T, 128), jnp.int32),
        pltpu.VMEM((L,), jnp.int32),
        pltpu.VMEM((T, L), jnp.float32),
        pltpu.VMEM((L,), jnp.float32),
        pltpu.VMEM((L,), jnp.float32),
        pltpu.VMEM_SHARED((4 * T, 128), jnp.float32),
        pltpu.SemaphoreType.DMA,
        pltpu.SemaphoreType.DMA,
    ],
)
def _loss_kernel(xt_hbm, mt_hbm, tgt_hbm, out,
                 xa, ma, thb, pth, tbuf, idxb, accv, avec, tvec,
                 shared, sem0, sem1):
    c = lax.axis_index("c")
    s = lax.axis_index("s")
    wid = s * NC + c
    tc = c * 4 + jnp.bitwise_and(s, 3)
    tcl = jnp.bitwise_and(s, 3)
    q = jnp.right_shift(s, 2)
    tbase = TPW * q
    col0 = 128 * tc

    pltpu.async_copy(
        xt_hbm.at[pl.ds(8 * tbase, 8 * TPW), pl.ds(col0, 128)],
        xa.at[pl.ds(0, 8 * TPW)], sem0)
    pltpu.async_copy(
        mt_hbm.at[pl.ds(8 * tbase, 8 * TPW), pl.ds(col0, 128)],
        ma.at[pl.ds(0, 8 * TPW)], sem1)
    t32 = jnp.where(q == 0, NTILES - 1, tbase)
    pltpu.async_copy(
        xt_hbm.at[pl.ds(8 * t32, 8), pl.ds(col0, 128)],
        xa.at[pl.ds(8 * TPW, 8)], sem0)
    pltpu.async_copy(
        mt_hbm.at[pl.ds(8 * t32, 8), pl.ds(col0, 128)],
        ma.at[pl.ds(8 * TPW, 8)], sem1)

    pltpu.sync_copy(tgt_hbm.at[pl.ds(0, 8), pl.ds(col0, 128)], tbuf.at[pl.ds(0, 8)])
    pltpu.sync_copy(tgt_hbm.at[pl.ds(8, 2), pl.ds(col0, 128)], tbuf.at[pl.ds(8, 2)])

    zero = jnp.zeros((L,), jnp.float32)

    def zero_body(t, carry):
        for sb in range(NSB):
            thb[t, pl.ds(L * sb, L)] = zero
        return carry

    lax.fori_loop(0, T, zero_body, jnp.int32(0))

    lanes = lax.iota(jnp.int32, L)
    idxb[...] = lanes + T * tcl

    def tgt_addr(t, sb):
        tg = tbuf[t, pl.ds(L * sb, L)]
        tr = jnp.right_shift(tg, 3)
        tl = tr - tbase
        extra = jnp.logical_and(tr == NTILES - 1, q == 0)
        valid = jnp.logical_or(jnp.logical_and(tl >= 0, tl < TPW), extra)
        idx = jnp.clip(jnp.where(extra, TPW, tl), 0, TPW)
        row = jnp.left_shift(idx, 3) + jnp.bitwise_and(tg, 7)
        return row, lanes + L * sb, valid

    pltpu.make_async_copy(
        xt_hbm.at[pl.ds(0, 8 * TPW), pl.ds(0, 128)],
        xa.at[pl.ds(0, 8 * TPW)], sem0).wait()
    pltpu.make_async_copy(
        xt_hbm.at[pl.ds(0, 8), pl.ds(0, 128)], xa.at[pl.ds(0, 8)], sem0).wait()

    def xg_body(t, carry):
        for sb in range(NSB):
            row, col, _ = tgt_addr(t, sb)
            pth[t, pl.ds(L * sb, L)] = plsc.load_gather(xa, [row, col])
        return carry

    lax.fori_loop(0, T, xg_body, jnp.int32(0))

    pltpu.make_async_copy(
        mt_hbm.at[pl.ds(0, 8 * TPW), pl.ds(0, 128)],
        ma.at[pl.ds(0, 8 * TPW)], sem1).wait()
    pltpu.make_async_copy(
        mt_hbm.at[pl.ds(0, 8), pl.ds(0, 128)], ma.at[pl.ds(0, 8)], sem1).wait()

    def mg_body(t, psum):
        for sb in range(NSB):
            row, col, valid = tgt_addr(t, sb)
            g = pth[t, pl.ds(L * sb, L)] - plsc.load_gather(ma, [row, col])
            psum = psum + jnp.where(valid, g, 0.0)
            pth[t, pl.ds(L * sb, L)] = jnp.where(valid, g - 1.0, 0.0)
        return psum

    psum = lax.fori_loop(0, T, mg_body, zero)

    @pl.when(q == 0)
    def _():
        pltpu.sync_copy(thb, shared.at[pl.ds(T * tcl, T)])
    plsc.subcore_barrier()
    pltpu.sync_copy(pth, shared.at[idxb.at[pl.ds(0, T)]], add=True)
    plsc.subcore_barrier()
    pltpu.sync_copy(shared.at[pl.ds(T * tcl, T)], thb)

    def tile_body(i, accs):
        base = jnp.left_shift(i, 3)

        def sb_body(sb, accs):
            accs = list(accs)
            ths = [thb[t, pl.ds(L * sb, L)] for t in range(T)]
            for r in range(8):
                s_ = xa[base + r, pl.ds(L * sb, L)] - ma[base + r, pl.ds(L * sb, L)]
                for t in range(T):
                    accs[t] = accs[t] + jnp.maximum(s_, ths[t])
            return tuple(accs)

        return lax.fori_loop(0, NSB, sb_body, accs)

    ntiles = jnp.where(q == 0, TPW + 1, TPW)
    accs = lax.fori_loop(0, ntiles, tile_body, tuple(zero for _ in range(T)))

    acc = accs[0]
    for t in range(1, T):
        acc = acc + accs[t]
    avec[...] = acc
    tvec[...] = psum
    pltpu.sync_copy(avec, out.at[wid])
    pltpu.sync_copy(tvec, out.at[NW + wid])


def kernel(input_data, target, adaptive_margin):
    out = _loss_kernel(
        input_data.T, adaptive_margin.T, target.astype(jnp.int32).T
    )
    bt = jnp.float32(B * T)
    total = (jnp.sum(out[:NW])
             - jnp.float32(C) * (jnp.sum(out[NW:]) - bt) - bt)
    return total / jnp.float32(C)
